# baseline (speedup 1.0000x reference)
import functools

import jax
import jax.numpy as jnp
from jax import lax
from jax.experimental import pallas as pl
from jax.experimental.pallas import tpu as pltpu
from jax.experimental.pallas import tpu_sc as plsc

BATCH = 4096
HIST = 200
EMBED = 64
HIDDEN = 128
OUT = 16

NUM_CORES = 2
NUM_SUBCORES = 16
NW = NUM_CORES * NUM_SUBCORES
BPW = BATCH // NW
CHUNK = 40
NCHUNK = HIST // CHUNK
NVEC = EMBED // 16

_mesh = plsc.VectorSubcoreMesh(core_axis_name="c", subcore_axis_name="s")


@functools.partial(
    pl.kernel,
    mesh=_mesh,
    compiler_params=pltpu.CompilerParams(use_tc_tiling_on_sc=False),
    out_type=jax.ShapeDtypeStruct((BATCH, EMBED), jnp.float32),
    scratch_types=[
        pltpu.VMEM((BPW, HIST), jnp.int32),
        pltpu.VMEM((HIST, EMBED), jnp.float32),
        pltpu.VMEM((HIST, EMBED), jnp.float32),
        pltpu.VMEM((BPW, EMBED), jnp.float32),
        pltpu.SemaphoreType.DMA,
        pltpu.SemaphoreType.DMA,
    ],
)
def _pool(x_hbm, table_hbm, out_hbm, xv, buf0, buf1, outbuf, sem0, sem1):
    wid = lax.axis_index("s") * NUM_CORES + lax.axis_index("c")
    base = wid * BPW

    pltpu.sync_copy(x_hbm.at[pl.ds(base, BPW)], xv)

    def chunk_copy(b, c, buf, sem):
        return pltpu.make_async_copy(
            table_hbm.at[xv.at[b, pl.ds(c * CHUNK, CHUNK)]],
            buf.at[pl.ds(c * CHUNK, CHUNK)],
            sem,
        )

    def start_row(b, buf, sem):
        for c in range(NCHUNK):
            chunk_copy(b, c, buf, sem).start()

    def wait_row(b, buf, sem):
        for c in range(NCHUNK):
            chunk_copy(b, c, buf, sem).wait()

    def acc_row(b, buf):
        def body(i, accs):
            l = i * 4
            new = []
            for j in range(NVEC):
                r0 = buf[l + 0, pl.ds(j * 16, 16)]
                r1 = buf[l + 1, pl.ds(j * 16, 16)]
                r2 = buf[l + 2, pl.ds(j * 16, 16)]
                r3 = buf[l + 3, pl.ds(j * 16, 16)]
                new.append(accs[j] + ((r0 + r1) + (r2 + r3)))
            return tuple(new)

        zero = jnp.zeros((16,), jnp.float32)
        accs = lax.fori_loop(0, HIST // 4, body, (zero,) * NVEC)
        scale = jnp.float32(1.0 / HIST)
        for j in range(NVEC):
            outbuf[b, pl.ds(j * 16, 16)] = accs[j] * scale

    start_row(0, buf0, sem0)
    start_row(1, buf1, sem1)

    def loop_body(i, carry):
        g = i * 2

        wait_row(g, buf0, sem0)
        acc_row(g, buf0)

        @pl.when(g + 2 < BPW)
        def _():
            start_row(g + 2, buf0, sem0)

        wait_row(g + 1, buf1, sem1)
        acc_row(g + 1, buf1)

        @pl.when(g + 3 < BPW)
        def _():
            start_row(g + 3, buf1, sem1)

        return carry

    lax.fori_loop(0, BPW // 2, loop_body, 0)

    pltpu.sync_copy(outbuf, out_hbm.at[pl.ds(base, BPW)])


def _mlp_body(p_ref, w1_ref, b1_ref, w2_ref, b2_ref, o_ref):
    h = jnp.dot(p_ref[...], w1_ref[...], preferred_element_type=jnp.float32)
    h = h + b1_ref[...]
    o_ref[...] = (
        jnp.dot(h, w2_ref[...], preferred_element_type=jnp.float32) + b2_ref[...]
    )


def kernel(x, table, W1, b1, W2, b2):
    x = x.astype(jnp.int32)
    pooled = _pool(x, table)
    return pl.pallas_call(
        _mlp_body,
        out_shape=jax.ShapeDtypeStruct((BATCH, OUT), jnp.float32),
    )(pooled, W1, b1.reshape(1, HIDDEN), W2, b2.reshape(1, OUT))
